# baseline (speedup 1.0000x reference)
import jax
import jax.numpy as jnp
import numpy as np
from jax.experimental import pallas as pl
from jax.experimental.pallas import tpu as pltpu

B, N, IN_DIM, OUT_DIM, HEADS = 4, 512, 128, 128, 4
C = OUT_DIM // HEADS
NEG_SLOPE = 0.2
JG = 4
NG = N // JG


def _gat_kernel(x_ref, xp_ref, adj_ref, wl_ref, bl_ref, wl4_ref, bl4_ref,
                wr4_ref, br4_ref, bd4_ref, att_col_ref, bias_ref,
                gamma_ref, beta_ref, out_ref):
    xb = x_ref[0]
    xl = jnp.dot(xb, wl_ref[...], preferred_element_type=jnp.float32)
    xl = xl + bl_ref[0]
    mask = adj_ref[0] != 0
    ones_col = jnp.ones((N, 1), jnp.float32)
    head_outs = []
    for h in range(HEADS):
        xl_h = xl[:, h * C:(h + 1) * C]
        u = jnp.dot(xl_h, att_col_ref[h * C:(h + 1) * C, :],
                    preferred_element_type=jnp.float32)
        xl4 = jnp.dot(xb, wl4_ref[h * IN_DIM:(h + 1) * IN_DIM, :],
                      preferred_element_type=jnp.float32)
        xl4 = xl4 + bl4_ref[0, h * C * JG:(h + 1) * C * JG]
        xri = br4_ref[0, h * C * JG:(h + 1) * C * JG]
        for js in range(JG):
            w_off = (h * JG + js) * IN_DIM
            xri = xri + jnp.dot(
                xp_ref[0, js * NG:(js + 1) * NG, :],
                wr4_ref[w_off:w_off + IN_DIM, :],
                preferred_element_type=jnp.float32)
        bd4 = bd4_ref[h * C * JG:(h + 1) * C * JG, :]
        strips = []
        for g in range(NG):
            az = jnp.abs(xl4 + xri[g:g + 1, :])
            strips.append(jax.lax.dot_general(
                az, bd4, (((1,), (0,)), ((), ())),
                preferred_element_type=jnp.float32))
        q = jnp.concatenate(strips, axis=1)
        q = q + u * jnp.float32(0.6)
        m = jnp.max(q, axis=0, keepdims=True)
        a = jnp.where(mask, jnp.exp(q - m), 0.0)
        xl_h1 = jnp.concatenate([xl_h, ones_col], axis=1)
        oh = jax.lax.dot_general(a, xl_h1, (((0,), (0,)), ((), ())),
                                 preferred_element_type=jnp.float32)
        denom = oh[:, C:C + 1]
        head_outs.append(oh[:, :C] / jnp.where(denom > 0, denom, 1.0))
    y = jnp.concatenate(head_outs, axis=1) + bias_ref[0]
    mean = jnp.mean(y, axis=1, keepdims=True)
    yc = y - mean
    var = jnp.mean(yc * yc, axis=1, keepdims=True)
    out_ref[0] = yc * jax.lax.rsqrt(var + 1e-5) * gamma_ref[0] + beta_ref[0]


@jax.jit
def kernel(x, adj, Wl, bl, Wr, br, att, bias, gamma, beta):
    f32 = jnp.float32
    U = np.zeros((JG, C, C * JG), dtype=np.float32)
    for js in range(JG):
        U[js, np.arange(C), np.arange(C) * JG + js] = 1.0
    U = jnp.asarray(U)
    R = jnp.sum(U, axis=0)
    wl4 = jnp.concatenate(
        [Wl[:, h * C:(h + 1) * C] @ R for h in range(HEADS)], axis=0)
    bl4 = jnp.concatenate(
        [bl[h * C:(h + 1) * C] @ R for h in range(HEADS)]).reshape(1, -1)
    wr4 = jnp.concatenate(
        [Wr[:, h * C:(h + 1) * C] @ U[js]
         for h in range(HEADS) for js in range(JG)], axis=0)
    br4 = jnp.concatenate(
        [br[h * C:(h + 1) * C] @ R for h in range(HEADS)]).reshape(1, -1)
    eye_pat = jnp.asarray(
        (np.arange(C * JG) % JG)[:, None] == np.arange(JG)[None, :],
        dtype=np.float32)
    bd4 = jnp.concatenate(
        [(0.4 * jnp.repeat(att[h], JG))[:, None] * eye_pat
         for h in range(HEADS)], axis=0)
    perm = (jnp.arange(N) % NG) * JG + jnp.arange(N) // NG
    xp = x[:, perm, :]
    row_spec = pl.BlockSpec((1, HEADS * C), lambda b: (0, 0))
    pk_spec = pl.BlockSpec((1, HEADS * C * JG), lambda b: (0, 0))
    out = pl.pallas_call(
        _gat_kernel,
        grid=(B,),
        in_specs=[
            pl.BlockSpec((1, N, IN_DIM), lambda b: (b, 0, 0)),
            pl.BlockSpec((1, N, IN_DIM), lambda b: (b, 0, 0)),
            pl.BlockSpec((1, N, N), lambda b: (b, 0, 0)),
            pl.BlockSpec((IN_DIM, HEADS * C), lambda b: (0, 0)),
            row_spec,
            pl.BlockSpec((HEADS * IN_DIM, C * JG), lambda b: (0, 0)),
            pk_spec,
            pl.BlockSpec((HEADS * JG * IN_DIM, C * JG), lambda b: (0, 0)),
            pk_spec,
            pl.BlockSpec((HEADS * C * JG, JG), lambda b: (0, 0)),
            pl.BlockSpec((HEADS * C, 1), lambda b: (0, 0)),
            row_spec,
            row_spec,
            row_spec,
        ],
        out_specs=pl.BlockSpec((1, N, OUT_DIM), lambda b: (b, 0, 0)),
        out_shape=jax.ShapeDtypeStruct((B, N, OUT_DIM), f32),
        compiler_params=pltpu.CompilerParams(
            dimension_semantics=("parallel",)),
    )(x, xp, adj, Wl, bl.reshape(1, -1), wl4, bl4, wr4, br4, bd4,
      att.reshape(-1, 1), bias.reshape(1, -1), gamma.reshape(1, -1),
      beta.reshape(1, -1))
    return out


# speedup vs baseline: 1.0439x; 1.0439x over previous
import jax
import jax.numpy as jnp
from jax.experimental import pallas as pl
from jax.experimental.pallas import tpu as pltpu

B, N, IN_DIM, OUT_DIM, HEADS = 4, 512, 128, 128, 4
C = OUT_DIM // HEADS
NEG_SLOPE = 0.2
TJ = N


def _gat_tile_kernel(x_ref, adj_ref, wl_ref, bl_ref, wr_ref, br_ref,
                     att_ref, att_col_ref, bias_ref, gamma_ref, beta_ref,
                     out_ref):
    xb = x_ref[0]
    xl = jnp.dot(xb, wl_ref[...], preferred_element_type=jnp.float32)
    xl = xl + bl_ref[0]
    xrt = jax.lax.dot_general(wr_ref[...], xb, (((0,), (1,)), ((), ())),
                              preferred_element_type=jnp.float32)
    mask = adj_ref[0] != 0
    ones_col = jnp.ones((N, 1), jnp.float32)
    head_outs = []
    for h in range(HEADS):
        xl_h = xl[:, h * C:(h + 1) * C]
        u = jnp.dot(xl_h, att_col_ref[h * C:(h + 1) * C, :],
                    preferred_element_type=jnp.float32)
        q = u * jnp.float32(0.6)
        for c in range(C):
            hc = h * C + c
            col = xl_h[:, c:c + 1] + br_ref[0, hc]
            z = col + xrt[hc:hc + 1, :]
            q = q + jnp.abs(z) * (att_ref[0, hc] * jnp.float32(0.4))
        m = jnp.max(q, axis=0, keepdims=True)
        a = jnp.where(mask, jnp.exp(q - m), 0.0)
        xl_h1 = jnp.concatenate([xl_h, ones_col], axis=1)
        oh = jax.lax.dot_general(a, xl_h1, (((0,), (0,)), ((), ())),
                                 preferred_element_type=jnp.float32)
        denom = oh[:, C:C + 1]
        head_outs.append(oh[:, :C] / jnp.where(denom > 0, denom, 1.0))
    y = jnp.concatenate(head_outs, axis=1) + bias_ref[0]
    mean = jnp.mean(y, axis=1, keepdims=True)
    yc = y - mean
    var = jnp.mean(yc * yc, axis=1, keepdims=True)
    out_ref[0] = yc * jax.lax.rsqrt(var + 1e-5) * gamma_ref[0] + beta_ref[0]


@jax.jit
def kernel(x, adj, Wl, bl, Wr, br, att, bias, gamma, beta):
    row_spec = pl.BlockSpec((1, HEADS * C), lambda b: (0, 0))
    out = pl.pallas_call(
        _gat_tile_kernel,
        grid=(B,),
        in_specs=[
            pl.BlockSpec((1, N, IN_DIM), lambda b: (b, 0, 0)),
            pl.BlockSpec((1, N, N), lambda b: (b, 0, 0)),
            pl.BlockSpec((IN_DIM, HEADS * C), lambda b: (0, 0)),
            row_spec,
            pl.BlockSpec((IN_DIM, HEADS * C), lambda b: (0, 0)),
            row_spec,
            row_spec,
            pl.BlockSpec((HEADS * C, 1), lambda b: (0, 0)),
            row_spec,
            row_spec,
            row_spec,
        ],
        out_specs=pl.BlockSpec((1, N, OUT_DIM), lambda b: (b, 0, 0)),
        out_shape=jax.ShapeDtypeStruct((B, N, OUT_DIM), jnp.float32),
        compiler_params=pltpu.CompilerParams(
            dimension_semantics=("parallel",)),
    )(x, adj, Wl, bl.reshape(1, -1), Wr, br.reshape(1, -1),
      att.reshape(1, -1), att.reshape(-1, 1), bias.reshape(1, -1),
      gamma.reshape(1, -1), beta.reshape(1, -1))
    return out
